# baseline (speedup 1.0000x reference)
import functools

import jax
import jax.numpy as jnp
from jax.experimental import pallas as pl
from jax.experimental.pallas import tpu as pltpu

RATIO_POS = 3
NUM_CLASSES = 21
B, A = 32, 8732
G = B * A
PACK = 8
MA = G // PACK
WA = PACK * NUM_CLASSES
BLK_A = 4096
MB = G // 128
MININT = -2147483648


def _stage_a(x_ref, lab_ref, sums_ref, chosen_ref):
    x = x_ref[...]
    lab = lab_ref[...].astype(jnp.float32)

    lane = jax.lax.broadcasted_iota(jnp.int32, (BLK_A, WA), 1)
    cls_of_lane = (lane % NUM_CLASSES).astype(jnp.float32)

    r = jax.lax.broadcasted_iota(jnp.int32, (WA, PACK), 0)
    c = jax.lax.broadcasted_iota(jnp.int32, (WA, PACK), 1)
    bd = (r // NUM_CLASSES == c).astype(jnp.float32)

    e = jnp.exp(x)
    sums_ref[...] = jnp.dot(e, bd, preferred_element_type=jnp.float32)

    lab_e = jnp.dot(lab, bd.T, preferred_element_type=jnp.float32)
    sel = jnp.where(cls_of_lane == lab_e, x, 0.0)
    chosen_ref[...] = jnp.dot(sel, bd, preferred_element_type=jnp.float32)


def _stage_b(sums_ref, chosen_ref, lab_ref, labf_ref, gt_ref, pr_ref,
             reg_ref, cls_ref):
    sums = sums_ref[...]
    chosen = chosen_ref[...]
    lab = lab_ref[...]
    labf = labf_ref[...]
    gt = gt_ref[...]
    pr = pr_ref[...]

    r4 = jax.lax.broadcasted_iota(jnp.int32, (128, 512), 0)
    c4 = jax.lax.broadcasted_iota(jnp.int32, (128, 512), 1)
    e4 = (c4 // 4 == r4).astype(jnp.float32)
    pos4 = jnp.dot(labf.astype(jnp.float32), e4,
                   preferred_element_type=jnp.float32) > 0.0

    d = pr - gt
    ad = jnp.abs(d)
    sl1 = jnp.where(ad < 1.0, 0.5 * d * d, ad - 0.5)
    box_loss = jnp.sum(jnp.where(pos4, sl1, 0.0))

    gt4 = jnp.dot(gt, e4.T, preferred_element_type=jnp.float32)
    num_pos = jnp.sum((gt4 > 0.0).astype(jnp.float32))

    pos = lab > 0
    posf = pos.astype(jnp.float32)
    conf = jnp.maximum(jnp.log(sums) - chosen, 0.0)
    sum_pos_conf = jnp.sum(conf * posf)

    aidx = jax.lax.broadcasted_iota(jnp.int32, (B, A), 1)
    keys = jnp.where(pos, -(aidx + 1),
                     jax.lax.bitcast_convert_type(conf, jnp.int32))

    t = RATIO_POS * jnp.sum(pos.astype(jnp.int32), axis=1, keepdims=True)

    def step(i, p):
        bit = jax.lax.shift_left(jnp.int32(1), jnp.int32(31) - i)
        cand = p | bit
        cnt = jnp.sum((keys >= (cand ^ MININT)).astype(jnp.int32),
                      axis=1, keepdims=True)
        return jnp.where(cnt >= t, cand, p)

    p = jax.lax.fori_loop(0, 32, step, jnp.zeros((B, 1), jnp.int32))
    theta = p ^ MININT

    gt_m = keys > theta
    eq_m = keys == theta
    c_gt = jnp.sum(gt_m.astype(jnp.float32), axis=1, keepdims=True)
    c_eq = jnp.sum(eq_m.astype(jnp.float32), axis=1, keepdims=True)
    s_gt = jnp.sum(jnp.where(gt_m, conf, 0.0), axis=1, keepdims=True)
    s_eq = jnp.sum(jnp.where(eq_m, conf, 0.0), axis=1, keepdims=True)
    tie = jnp.where(c_eq > 0.0,
                    (t.astype(jnp.float32) - c_gt) * s_eq
                    / jnp.where(c_eq > 0.0, c_eq, 1.0),
                    0.0)
    s_bg = jnp.sum(s_gt + tie)

    reg_ref[0] = box_loss / num_pos
    cls_ref[0] = (sum_pos_conf + s_bg) / num_pos


@functools.partial(jax.jit, static_argnames=("interpret",))
def kernel(gt_bboxes, gt_labels, pred_bboxes, pred_labels, interpret=False):
    lab32 = gt_labels.astype(jnp.int32)
    xf = pred_labels.reshape(MA, WA)
    labr = lab32.reshape(MA, PACK)
    labf = lab32.reshape(MB, 128)
    gtf = gt_bboxes.reshape(MB, 512)
    prf = pred_bboxes.reshape(MB, 512)

    na = pl.cdiv(MA, BLK_A)
    sums, chosen = pl.pallas_call(
        _stage_a,
        grid=(na,),
        in_specs=[
            pl.BlockSpec((BLK_A, WA), lambda i: (i, 0)),
            pl.BlockSpec((BLK_A, PACK), lambda i: (i, 0)),
        ],
        out_specs=[
            pl.BlockSpec((BLK_A, PACK), lambda i: (i, 0)),
            pl.BlockSpec((BLK_A, PACK), lambda i: (i, 0)),
        ],
        out_shape=[
            jax.ShapeDtypeStruct((MA, PACK), jnp.float32),
            jax.ShapeDtypeStruct((MA, PACK), jnp.float32),
        ],
        interpret=interpret,
    )(xf, labr)

    reg, cls = pl.pallas_call(
        _stage_b,
        in_specs=[
            pl.BlockSpec((B, A), lambda: (0, 0)),
            pl.BlockSpec((B, A), lambda: (0, 0)),
            pl.BlockSpec((B, A), lambda: (0, 0)),
            pl.BlockSpec((MB, 128), lambda: (0, 0)),
            pl.BlockSpec((MB, 512), lambda: (0, 0)),
            pl.BlockSpec((MB, 512), lambda: (0, 0)),
        ],
        out_specs=[
            pl.BlockSpec(memory_space=pltpu.SMEM),
            pl.BlockSpec(memory_space=pltpu.SMEM),
        ],
        out_shape=[
            jax.ShapeDtypeStruct((1,), jnp.float32),
            jax.ShapeDtypeStruct((1,), jnp.float32),
        ],
        interpret=interpret,
    )(sums.reshape(B, A), chosen.reshape(B, A), lab32, labf, gtf, prf)
    return (reg[0], cls[0])


# speedup vs baseline: 3.9890x; 3.9890x over previous
import functools

import jax
import jax.numpy as jnp
from jax.experimental import pallas as pl
from jax.experimental.pallas import tpu as pltpu

RATIO_POS = 3
NUM_CLASSES = 21
B, A = 32, 8732
MININT = -2147483648


def _ssd_kernel(gt_ref, pr_ref, lab_ref, logit_ref, reg_ref, cls_ref,
                conf_s, key_s, acc_s):
    b = pl.program_id(0)

    @pl.when(b == 0)
    def _init():
        acc_s[0] = 0.0
        acc_s[1] = 0.0
        acc_s[2] = 0.0

    gt = gt_ref[0]
    pr = pr_ref[0]
    lab = lab_ref[0]
    x = jnp.transpose(logit_ref[0], (1, 0))

    pos = lab > 0
    posf = pos.astype(jnp.float32)

    d = pr - gt
    ad = jnp.abs(d)
    sl1 = jnp.where(ad < 1.0, 0.5 * d * d, ad - 0.5)
    acc_s[0] += jnp.sum(jnp.sum(sl1, axis=0, keepdims=True) * posf)

    acc_s[1] += jnp.sum((jnp.sum(gt, axis=0, keepdims=True) > 0)
                        .astype(jnp.float32))

    lse = jnp.log(jnp.sum(jnp.exp(x), axis=0, keepdims=True))
    cls_iota = jax.lax.broadcasted_iota(jnp.int32, x.shape, 0)
    chosen = jnp.sum(jnp.where(cls_iota == lab, x, 0.0), axis=0,
                     keepdims=True)
    conf = jnp.maximum(lse - chosen, 0.0)

    acc_s[2] += jnp.sum(conf * posf)

    aidx = jax.lax.broadcasted_iota(jnp.int32, (1, A), 1)
    confbits = jax.lax.bitcast_convert_type(conf, jnp.int32)
    key = jnp.where(pos, -(aidx + 1), confbits)

    conf_s[pl.ds(b, 1), :] = conf
    key_s[pl.ds(b, 1), :] = key

    @pl.when(b == pl.num_programs(0) - 1)
    def _mine():
        keys = key_s[:, :]
        confs = conf_s[:, :]
        t = RATIO_POS * jnp.sum((keys < 0).astype(jnp.int32), axis=1,
                                keepdims=True)

        def step(i, p):
            bit = jax.lax.shift_left(jnp.int32(1), jnp.int32(31) - i)
            cand = p | bit
            cnt = jnp.sum((keys >= (cand ^ MININT)).astype(jnp.int32),
                          axis=1, keepdims=True)
            return jnp.where(cnt >= t, cand, p)

        p = jax.lax.fori_loop(0, 32, step, jnp.zeros((B, 1), jnp.int32))
        theta = p ^ MININT

        gt_m = keys > theta
        eq_m = keys == theta
        c_gt = jnp.sum(gt_m.astype(jnp.float32), axis=1, keepdims=True)
        c_eq = jnp.sum(eq_m.astype(jnp.float32), axis=1, keepdims=True)
        s_gt = jnp.sum(jnp.where(gt_m, confs, 0.0), axis=1, keepdims=True)
        s_eq = jnp.sum(jnp.where(eq_m, confs, 0.0), axis=1, keepdims=True)
        tie = jnp.where(c_eq > 0.0,
                        (t.astype(jnp.float32) - c_gt) * s_eq
                        / jnp.where(c_eq > 0.0, c_eq, 1.0),
                        0.0)
        s_bg = jnp.sum(s_gt + tie)

        num_pos = acc_s[1]
        reg_ref[0] = acc_s[0] / num_pos
        cls_ref[0] = (acc_s[2] + s_bg) / num_pos


@functools.partial(jax.jit, static_argnames=("interpret",))
def kernel(gt_bboxes, gt_labels, pred_bboxes, pred_labels, interpret=False):
    gt_t = jnp.transpose(gt_bboxes, (0, 2, 1))
    pr_t = jnp.transpose(pred_bboxes, (0, 2, 1))
    lab3 = gt_labels.reshape(B, 1, A).astype(jnp.int32)

    reg, cls = pl.pallas_call(
        _ssd_kernel,
        grid=(B,),
        in_specs=[
            pl.BlockSpec((1, 4, A), lambda b: (b, 0, 0)),
            pl.BlockSpec((1, 4, A), lambda b: (b, 0, 0)),
            pl.BlockSpec((1, 1, A), lambda b: (b, 0, 0)),
            pl.BlockSpec((1, A, NUM_CLASSES), lambda b: (b, 0, 0)),
        ],
        out_specs=[
            pl.BlockSpec(memory_space=pltpu.SMEM),
            pl.BlockSpec(memory_space=pltpu.SMEM),
        ],
        out_shape=[
            jax.ShapeDtypeStruct((1,), jnp.float32),
            jax.ShapeDtypeStruct((1,), jnp.float32),
        ],
        scratch_shapes=[
            pltpu.VMEM((B, A), jnp.float32),
            pltpu.VMEM((B, A), jnp.int32),
            pltpu.SMEM((4,), jnp.float32),
        ],
        interpret=interpret,
    )(gt_t, pr_t, lab3, pred_labels)
    return (reg[0], cls[0])


# speedup vs baseline: 6.8401x; 1.7147x over previous
import functools

import jax
import jax.numpy as jnp
from jax.experimental import pallas as pl
from jax.experimental.pallas import tpu as pltpu

RATIO_POS = 3
NUM_CLASSES = 21
B, A = 32, 8732
MININT = -2147483648


def _box_kernel(gt_ref, pr_ref, lab_ref, box_ref, acc_s):
    b = pl.program_id(0)

    @pl.when(b == 0)
    def _init():
        acc_s[0] = 0.0
        acc_s[1] = 0.0

    gt = gt_ref[0]
    pr = pr_ref[0]
    lab = lab_ref[0]

    posf = (lab > 0).astype(jnp.float32)
    d = pr - gt
    ad = jnp.abs(d)
    sl1 = jnp.where(ad < 1.0, 0.5 * d * d, ad - 0.5)
    acc_s[0] += jnp.sum(jnp.sum(sl1, axis=0, keepdims=True) * posf)
    acc_s[1] += jnp.sum((jnp.sum(gt, axis=0, keepdims=True) > 0)
                        .astype(jnp.float32))

    @pl.when(b == pl.num_programs(0) - 1)
    def _out():
        box_ref[0] = acc_s[0]
        box_ref[1] = acc_s[1]


def _conf_kernel(lab_ref, logit_ref, box_ref, reg_ref, cls_ref,
                 conf_s, key_s, acc_s):
    b = pl.program_id(0)

    @pl.when(b == 0)
    def _init():
        acc_s[0] = 0.0

    lab = lab_ref[0]
    x = logit_ref[0]

    pos = lab > 0

    lse = jnp.log(jnp.sum(jnp.exp(x), axis=0, keepdims=True))
    cls_iota = jax.lax.broadcasted_iota(jnp.int32, x.shape, 0)
    chosen = jnp.sum(jnp.where(cls_iota == lab, x, 0.0), axis=0,
                     keepdims=True)
    conf = jnp.maximum(lse - chosen, 0.0)

    acc_s[0] += jnp.sum(conf * pos.astype(jnp.float32))

    aidx = jax.lax.broadcasted_iota(jnp.int32, (1, A), 1)
    confbits = jax.lax.bitcast_convert_type(conf, jnp.int32)
    key = jnp.where(pos, -(aidx + 1), confbits)

    conf_s[pl.ds(b, 1), :] = conf
    key_s[pl.ds(b, 1), :] = key

    @pl.when(b == pl.num_programs(0) - 1)
    def _mine():
        keys = key_s[:, :]
        confs = conf_s[:, :]
        t = RATIO_POS * jnp.sum((keys < 0).astype(jnp.int32), axis=1,
                                keepdims=True)

        def step(i, p):
            bit = jax.lax.shift_left(jnp.int32(1), jnp.int32(31) - i)
            cand = p | bit
            cnt = jnp.sum((keys >= (cand ^ MININT)).astype(jnp.int32),
                          axis=1, keepdims=True)
            return jnp.where(cnt >= t, cand, p)

        p = jax.lax.fori_loop(0, 32, step, jnp.zeros((B, 1), jnp.int32))
        theta = p ^ MININT

        gt_m = keys > theta
        eq_m = keys == theta
        c_gt = jnp.sum(gt_m.astype(jnp.float32), axis=1, keepdims=True)
        c_eq = jnp.sum(eq_m.astype(jnp.float32), axis=1, keepdims=True)
        s_gt = jnp.sum(jnp.where(gt_m, confs, 0.0), axis=1, keepdims=True)
        s_eq = jnp.sum(jnp.where(eq_m, confs, 0.0), axis=1, keepdims=True)
        tie = jnp.where(c_eq > 0.0,
                        (t.astype(jnp.float32) - c_gt) * s_eq
                        / jnp.where(c_eq > 0.0, c_eq, 1.0),
                        0.0)
        s_bg = jnp.sum(s_gt + tie)

        num_pos = box_ref[1]
        reg_ref[0] = box_ref[0] / num_pos
        cls_ref[0] = (acc_s[0] + s_bg) / num_pos


@functools.partial(jax.jit, static_argnames=("interpret",))
def kernel(gt_bboxes, gt_labels, pred_bboxes, pred_labels, interpret=False):
    gt_t = jnp.transpose(gt_bboxes, (0, 2, 1))
    pr_t = jnp.transpose(pred_bboxes, (0, 2, 1))
    lab3 = gt_labels.reshape(B, 1, A).astype(jnp.int32)
    logit_t = jnp.transpose(pred_labels, (0, 2, 1))

    box = pl.pallas_call(
        _box_kernel,
        grid=(B,),
        in_specs=[
            pl.BlockSpec((1, 4, A), lambda b: (b, 0, 0)),
            pl.BlockSpec((1, 4, A), lambda b: (b, 0, 0)),
            pl.BlockSpec((1, 1, A), lambda b: (b, 0, 0)),
        ],
        out_specs=pl.BlockSpec(memory_space=pltpu.SMEM),
        out_shape=jax.ShapeDtypeStruct((2,), jnp.float32),
        scratch_shapes=[pltpu.SMEM((2,), jnp.float32)],
        interpret=interpret,
    )(gt_t, pr_t, lab3)

    reg, cls = pl.pallas_call(
        _conf_kernel,
        grid=(B,),
        in_specs=[
            pl.BlockSpec((1, 1, A), lambda b: (b, 0, 0)),
            pl.BlockSpec((1, NUM_CLASSES, A), lambda b: (b, 0, 0)),
            pl.BlockSpec(memory_space=pltpu.SMEM),
        ],
        out_specs=[
            pl.BlockSpec(memory_space=pltpu.SMEM),
            pl.BlockSpec(memory_space=pltpu.SMEM),
        ],
        out_shape=[
            jax.ShapeDtypeStruct((1,), jnp.float32),
            jax.ShapeDtypeStruct((1,), jnp.float32),
        ],
        scratch_shapes=[
            pltpu.VMEM((B, A), jnp.float32),
            pltpu.VMEM((B, A), jnp.int32),
            pltpu.SMEM((1,), jnp.float32),
        ],
        interpret=interpret,
    )(lab3, logit_t, box)
    return (reg[0], cls[0])
